# baseline (speedup 1.0000x reference)
import functools

import jax
import jax.numpy as jnp
from jax import lax
from jax.experimental import pallas as pl
from jax.experimental.pallas import tpu as pltpu
from jax.experimental.pallas import tpu_sc as plsc

_GAMMA = 12.0
_D = 64
_HD = _D // 2
_B = 16384
_NC = 2
_NS = 16
_L = 16
_NW = _NC * _NS
_BPW = _B // _NW
_CH = 128
_NCH = _BPW // _CH
_GPC = _CH // _L

_mesh = plsc.VectorSubcoreMesh(core_axis_name="c", subcore_axis_name="s")


@functools.partial(
    pl.kernel,
    out_type=jax.ShapeDtypeStruct((_B,), jnp.float32),
    mesh=_mesh,
    scratch_types=[
        pltpu.VMEM((_NCH, _CH), jnp.int32),
        pltpu.VMEM((_NCH, _CH), jnp.int32),
        pltpu.VMEM((_NCH, _CH), jnp.int32),
        pltpu.VMEM((_CH, _HD), jnp.float32),
        pltpu.VMEM((_CH, _HD), jnp.float32),
        pltpu.VMEM((_CH, _HD), jnp.float32),
        pltpu.VMEM((_CH, _HD), jnp.float32),
        pltpu.VMEM((_CH, _HD), jnp.float32),
        pltpu.VMEM((_CH, _HD), jnp.float32),
        pltpu.VMEM((_CH, _HD), jnp.float32),
        pltpu.VMEM((_CH, _HD), jnp.float32),
        pltpu.VMEM((_CH, _D), jnp.float32),
        pltpu.VMEM((_CH, _D), jnp.float32),
        pltpu.VMEM((_BPW,), jnp.float32),
        pltpu.SemaphoreType.DMA,
        pltpu.SemaphoreType.DMA,
    ],
    compiler_params=pltpu.CompilerParams(
        needs_layout_passes=False, use_tc_tiling_on_sc=False),
)
def _kge_score(entlo_hbm, enthi_hbm, relemb_hbm, head_hbm, rel_hbm, tail_hbm,
               out_hbm, idx_h, idx_r, idx_t,
               hl0, hl1, hh0, hh1, tl0, tl1, th0, th1, r0, r1,
               o_v, sem0, sem1):
    wid = lax.axis_index("s") * _NC + lax.axis_index("c")
    base = wid * _BPW

    pltpu.sync_copy(head_hbm.at[wid], idx_h)
    pltpu.sync_copy(rel_hbm.at[wid], idx_r)
    pltpu.sync_copy(tail_hbm.at[wid], idx_t)

    bufs = ((hl0, hh0, tl0, th0, r0, sem0), (hl1, hh1, tl1, th1, r1, sem1))

    def fire(c, hl, hh, tl, th, rb, sem):
        return (
            pltpu.async_copy(entlo_hbm.at[idx_h.at[c]], hl, sem),
            pltpu.async_copy(enthi_hbm.at[idx_h.at[c]], hh, sem),
            pltpu.async_copy(entlo_hbm.at[idx_t.at[c]], tl, sem),
            pltpu.async_copy(enthi_hbm.at[idx_t.at[c]], th, sem),
            pltpu.async_copy(relemb_hbm.at[idx_r.at[c]], rb, sem),
        )

    lanes = lax.iota(jnp.int32, _L)
    pend = fire(0, *bufs[0])

    for c in range(_NCH):
        for cp in pend:
            cp.wait()
        if c + 1 < _NCH:
            pend = fire(c + 1, *bufs[(c + 1) % 2])
        hl, hh, tl, th, rb, _ = bufs[c % 2]

        def group(g, carry):
            tots = []
            for row in range(_L):
                i = g * _L + row
                s = jnp.zeros((_L,), jnp.float32)
                for cc in range(_HD // _L):
                    sl = pl.ds(cc * _L, _L)
                    rsl = pl.ds(cc * _L, _L)
                    dlo = hl[i, sl] + rb[i, rsl] - tl[i, sl]
                    s = s + dlo * dlo
                    rsl2 = pl.ds(_HD + cc * _L, _L)
                    dhi = hh[i, sl] + rb[i, rsl2] - th[i, sl]
                    s = s + dhi * dhi
                tot = lax.reduce_sum_p.bind(s, axes=(0,))
                tots.append(jnp.where(lanes == row, tot, 0.0))
            while len(tots) > 1:
                tots = [a + b for a, b in zip(tots[::2], tots[1::2])]
            x = tots[0] + 1e-12
            seed = plsc.bitcast(
                (plsc.bitcast(x, jnp.int32) >> 1) + 0x1FBD1DF5, jnp.float32)
            y = 0.5 * (seed + x / seed)
            y = 0.5 * (y + x / y)
            o_v[pl.ds(c * _CH + g * _L, _L)] = _GAMMA - y
            return carry

        lax.fori_loop(0, _GPC, group, 0)

    pltpu.sync_copy(o_v, out_hbm.at[pl.ds(base, _BPW)])


def kernel(entity_emb, relation_emb, head, rel, tail):
    ent_lo = entity_emb[:, :_HD]
    ent_hi = entity_emb[:, _HD:]
    head3 = head.reshape(_NW, _NCH, _CH)
    rel3 = rel.reshape(_NW, _NCH, _CH)
    tail3 = tail.reshape(_NW, _NCH, _CH)
    return _kge_score(ent_lo, ent_hi, relation_emb, head3, rel3, tail3)


# speedup vs baseline: 1.8385x; 1.8385x over previous
import functools

import jax
import jax.numpy as jnp
from jax import lax
from jax.experimental import pallas as pl
from jax.experimental.pallas import tpu as pltpu
from jax.experimental.pallas import tpu_sc as plsc

_GAMMA = 12.0
_D = 64
_B = 16384
_NC = 2
_NS = 16
_L = 16
_NW = _NC * _NS
_BPW = _B // _NW
_CH = 128
_NCH = _BPW // _CH
_GPC = _CH // _L

_mesh = plsc.VectorSubcoreMesh(core_axis_name="c", subcore_axis_name="s")


@functools.partial(
    pl.kernel,
    out_type=jax.ShapeDtypeStruct((_B,), jnp.float32),
    mesh=_mesh,
    scratch_types=[
        pltpu.VMEM((_BPW,), jnp.int32),
        pltpu.VMEM((_BPW,), jnp.int32),
        pltpu.VMEM((_BPW,), jnp.int32),
        pltpu.VMEM((_CH, _D), jnp.float32),
        pltpu.VMEM((_CH, _D), jnp.float32),
        pltpu.VMEM((_CH, _D), jnp.float32),
        pltpu.VMEM((_CH, _D), jnp.float32),
        pltpu.VMEM((_CH, _D), jnp.float32),
        pltpu.VMEM((_CH, _D), jnp.float32),
        pltpu.VMEM((_BPW,), jnp.float32),
        pltpu.SemaphoreType.DMA,
        pltpu.SemaphoreType.DMA,
    ],
    compiler_params=pltpu.CompilerParams(
        needs_layout_passes=False, use_tc_tiling_on_sc=False),
)
def _kge_score(ent_hbm, relemb_hbm, head_hbm, rel_hbm, tail_hbm, out_hbm,
               idx_h, idx_r, idx_t, h0, h1, r0, r1, t0, t1, o_v, sem0, sem1):
    wid = lax.axis_index("s") * _NC + lax.axis_index("c")
    base = wid * _BPW

    pltpu.sync_copy(head_hbm.at[pl.ds(base, _BPW)], idx_h)
    pltpu.sync_copy(rel_hbm.at[pl.ds(base, _BPW)], idx_r)
    pltpu.sync_copy(tail_hbm.at[pl.ds(base, _BPW)], idx_t)

    bufs = ((h0, r0, t0, sem0), (h1, r1, t1, sem1))

    def fire(c, hb, rb, tb, sem):
        sl = pl.ds(c * _CH, _CH)
        return (
            pltpu.async_copy(ent_hbm.at[idx_h.at[sl]], hb, sem),
            pltpu.async_copy(relemb_hbm.at[idx_r.at[sl]], rb, sem),
            pltpu.async_copy(ent_hbm.at[idx_t.at[sl]], tb, sem),
        )

    lanes = lax.iota(jnp.int32, _L)
    pend = fire(0, *bufs[0])

    for c in range(_NCH):
        for cp in pend:
            cp.wait()
        if c + 1 < _NCH:
            pend = fire(c + 1, *bufs[(c + 1) % 2])
        hb, rb, tb, _ = bufs[c % 2]

        def group(g, carry):
            tots = []
            for row in range(_L):
                i = g * _L + row
                s = jnp.zeros((_L,), jnp.float32)
                for cc in range(_D // _L):
                    sl = pl.ds(cc * _L, _L)
                    diff = hb[i, sl] + rb[i, sl] - tb[i, sl]
                    s = s + diff * diff
                tot = lax.reduce_sum_p.bind(s, axes=(0,))
                tots.append(jnp.where(lanes == row, tot, 0.0))
            while len(tots) > 1:
                tots = [a + b for a, b in zip(tots[::2], tots[1::2])]
            x = tots[0] + 1e-12
            seed = plsc.bitcast(
                (plsc.bitcast(x, jnp.int32) >> 1) + 0x1FBD1DF5, jnp.float32)
            y = 0.5 * (seed + x / seed)
            y = 0.5 * (y + x / y)
            o_v[pl.ds(c * _CH + g * _L, _L)] = _GAMMA - y
            return carry

        lax.fori_loop(0, _GPC, group, 0)

    pltpu.sync_copy(o_v, out_hbm.at[pl.ds(base, _BPW)])


def kernel(entity_emb, relation_emb, head, rel, tail):
    return _kge_score(entity_emb, relation_emb, head, rel, tail)
